# baseline (speedup 1.0000x reference)
import functools

import jax
import jax.numpy as jnp
from jax.experimental import pallas as pl
from jax.experimental.pallas import tpu as pltpu


def _bb_body(e_ref, w1_ref, b1_ref, w2_ref, b2_ref, w3_ref, b3_ref, o_ref):
    h = jnp.maximum(e_ref[...] @ w1_ref[...] + b1_ref[...], 0.0)
    h = jnp.maximum(h @ w2_ref[...] + b2_ref[...], 0.0)
    o_ref[...] = h @ w3_ref[...] + b3_ref[...]


def _backbone(embs, bp):
    e = jnp.pad(embs, ((0, 7), (0, 0)))
    out = pl.pallas_call(
        _bb_body,
        out_shape=jax.ShapeDtypeStruct((8, 256), jnp.float32),
    )(e, bp['W1'], bp['b1'].reshape(1, -1), bp['W2'], bp['b2'].reshape(1, -1),
      bp['W3'], bp['b3'].reshape(1, -1))
    return out[:1]


def _gatv2(x, src, dst, p, n):
    xl = x @ p['Wl']
    xr = x @ p['Wr']
    m = xl[src] + xr[dst]
    e = jax.nn.leaky_relu(m, 0.2)
    logits = e @ p['att']
    mx = jax.ops.segment_max(logits, dst, num_segments=n)
    ex = jnp.exp(logits - mx[dst])
    den = jax.ops.segment_sum(ex, dst, num_segments=n)
    alpha = ex / (den[dst] + 1e-16)
    out = jax.ops.segment_sum(xl[src] * alpha[:, None], dst, num_segments=n)
    return out + p['b']


def _ast_embed(x, edge_index, root, layers):
    n = x.shape[0]
    loop = jnp.arange(n, dtype=edge_index.dtype)
    src = jnp.concatenate([edge_index[0], loop])
    dst = jnp.concatenate([edge_index[1], loop])
    h = x
    for i, p in enumerate(layers):
        h = _gatv2(h, src, dst, p, n)
        if i < len(layers) - 1:
            h = jax.nn.relu(h)
    return h[root]


def kernel(x_lhs, edge_index_lhs, lhs_root, x_rhs, edge_index_rhs, rhs_root,
           x_sketch, edge_index_sketch, sketch_root, params):
    lhs_emb = _ast_embed(x_lhs, edge_index_lhs, lhs_root, params['lhs'])
    rhs_emb = _ast_embed(x_rhs, edge_index_rhs, rhs_root, params['rhs'])
    sketch_emb = _ast_embed(x_sketch, edge_index_sketch, sketch_root, params['sketch'])
    embs = jnp.concatenate([sketch_emb, lhs_emb, rhs_emb], axis=-1)
    return _backbone(embs, params['backbone'])


# speedup vs baseline: 4.6872x; 4.6872x over previous
import functools

import jax
import jax.numpy as jnp
from jax import lax
from jax.experimental import pallas as pl
from jax.experimental.pallas import tpu as pltpu
from jax.experimental.pallas import tpu_sc as plsc

N = 10000
M = 30000
D = 256
NJ = D // 16
RANGE = 240
NRANGES = M // RANGE
CH = 64
EE = 3 * (320000 + N)
EEP = 990080
NSL = 336
NOFF_PAD = M + NSL + 16

_SENTINEL = 1 << 30


def _mm_body_nobias(h_ref, wl_ref, wr_ref, att_ref, xl_ref, xr_ref, c_ref):
    x = h_ref[0]
    xl = jnp.dot(x, wl_ref[0], preferred_element_type=jnp.float32)
    xr = jnp.dot(x, wr_ref[0], preferred_element_type=jnp.float32)
    xl_ref[0] = xl
    xr_ref[0] = xr
    t = xl + xr
    e = jnp.maximum(t, 0.2 * t)
    c_ref[0, 0] = jnp.sum(e * att_ref[0], axis=1)


def _mm_body_bias(h_ref, b_ref, wl_ref, wr_ref, att_ref, xl_ref, xr_ref, c_ref):
    x = jnp.maximum(h_ref[0] + b_ref[0], 0.0)
    xl = jnp.dot(x, wl_ref[0], preferred_element_type=jnp.float32)
    xr = jnp.dot(x, wr_ref[0], preferred_element_type=jnp.float32)
    xl_ref[0] = xl
    xr_ref[0] = xr
    t = xl + xr
    e = jnp.maximum(t, 0.2 * t)
    c_ref[0, 0] = jnp.sum(e * att_ref[0], axis=1)


def _layer_mm(h, b_prev, wl, wr, att):
    din = h.shape[-1]
    nb = N // 400
    grid = (3, nb)
    h_spec = pl.BlockSpec((1, 400, din), lambda g, i: (g, i, 0))
    w_spec = pl.BlockSpec((1, din, D), lambda g, i: (g, 0, 0))
    att_spec = pl.BlockSpec((1, 1, D), lambda g, i: (g, 0, 0))
    b_spec = pl.BlockSpec((1, 1, D), lambda g, i: (g, 0, 0))
    out_specs = [
        pl.BlockSpec((1, 400, D), lambda g, i: (g, i, 0)),
        pl.BlockSpec((1, 400, D), lambda g, i: (g, i, 0)),
        pl.BlockSpec((1, 1, 400), lambda g, i: (g * nb + i, 0, 0)),
    ]
    out_shape = [
        jax.ShapeDtypeStruct((3, N, D), jnp.float32),
        jax.ShapeDtypeStruct((3, N, D), jnp.float32),
        jax.ShapeDtypeStruct((3 * nb, 1, 400), jnp.float32),
    ]
    att3 = att.reshape(3, 1, D)
    if b_prev is None:
        xl, xr, c = pl.pallas_call(
            _mm_body_nobias,
            grid=grid,
            in_specs=[h_spec, w_spec, w_spec, att_spec],
            out_specs=out_specs,
            out_shape=out_shape,
        )(h, wl, wr, att3)
    else:
        xl, xr, c = pl.pallas_call(
            _mm_body_bias,
            grid=grid,
            in_specs=[h_spec, b_spec, w_spec, w_spec, att_spec],
            out_specs=out_specs,
            out_shape=out_shape,
        )(h, b_prev.reshape(3, 1, D), wl, wr, att3)
    return xl.reshape(M, D), xr.reshape(M, D), c.reshape(M)


def _edge_body(xl_hbm, xr_hbm, c_hbm, src_hbm, noff_hbm, att_hbm, out_hbm,
               xr_v, xl_v, src_v, noff_v, c_v, att_v, gsem):
    cid = lax.axis_index("c")
    sid = lax.axis_index("s")
    wid = sid * 2 + cid

    pltpu.sync_copy(att_hbm, att_v)

    def do_range(r):
        r0 = r * RANGE
        pltpu.sync_copy(xr_hbm.at[pl.ds(r0 * D, RANGE * D)],
                        xr_v.at[pl.ds(0, RANGE * D)])
        pltpu.sync_copy(noff_hbm.at[pl.ds(r0, NSL)], noff_v)
        pltpu.sync_copy(c_hbm.at[pl.ds(r0, NSL)], c_v)

        head = noff_v[pl.ds(0, 16)]
        estart = head[0]
        eend = noff_v[pl.ds(RANGE, 16)][0]
        c0 = lax.shift_right_logical(estart, 6)
        c1 = lax.shift_right_logical(eend, 6) + 1

        g0 = (r0 >= N).astype(jnp.int32) + (r0 >= 2 * N).astype(jnp.int32)
        zero16 = jnp.zeros((16,), jnp.float32)
        init = (r0, g0, head[1], c_v[pl.ds(0, 16)][0], zero16) + tuple(
            zero16 for _ in range(NJ))

        def chunk_body(ci, carry):
            ebase = ci * CH
            pltpu.sync_copy(src_hbm.at[pl.ds(ebase, CH)], src_v)
            pltpu.async_copy(xl_hbm.at[src_v], xl_v, gsem).wait()

            def edge_body(e, ec):
                v, g, enext, cv, den = ec[0], ec[1], ec[2], ec[3], ec[4]
                accs = ec[5:]
                eg = ebase + e

                def fin(op):
                    v_, g_, enext_, cv_, den_ = op[0], op[1], op[2], op[3], op[4]
                    accs_ = op[5:]
                    invd = 1.0 / (den_ + 1e-16)
                    sidx = jnp.minimum(v_ - r0, RANGE)
                    sbase = sidx * D
                    for j in range(NJ):
                        xr_v[pl.ds(sbase + 16 * j, 16)] = accs_[j] * invd
                    v2 = v_ + 1
                    g2 = ((v2 >= N).astype(jnp.int32)
                          + (v2 >= 2 * N).astype(jnp.int32))
                    enext2 = noff_v[pl.ds(v2 - r0 + 1, 16)][0]
                    cv2 = c_v[pl.ds(v2 - r0, 16)][0]
                    return (v2, g2, enext2, cv2, zero16) + tuple(
                        zero16 for _ in range(NJ))

                ec = lax.cond(eg == enext, fin, lambda op: op, ec)
                v, g, enext, cv, den = ec[0], ec[1], ec[2], ec[3], ec[4]
                accs = ec[5:]

                xbase = jnp.minimum(v - r0, RANGE - 1) * D
                abase = g * D
                p = zero16
                xls = []
                for j in range(NJ):
                    xlj = xl_v[e, pl.ds(16 * j, 16)]
                    xls.append(xlj)
                    xrj = xr_v[pl.ds(xbase + 16 * j, 16)]
                    t = xlj + xrj
                    lr = jnp.maximum(t, 0.2 * t)
                    p = p + att_v[pl.ds(abase + 16 * j, 16)] * lr
                lanes = lax.iota(jnp.int32, 16)
                for k in (8, 4, 2, 1):
                    p = p + p.at[lanes ^ k].get(mode='promise_in_bounds')
                inr = jnp.logical_and(eg >= estart, eg < eend)
                wv = jnp.exp(p - cv)
                wv = jnp.where(inr, wv, 0.0)
                den = den + wv
                accs = tuple(accs[j] + wv * xls[j] for j in range(NJ))
                return (v, g, enext, cv, den) + accs

            return lax.fori_loop(0, CH, edge_body, carry)

        lax.fori_loop(c0, c1, chunk_body, init)
        pltpu.sync_copy(xr_v.at[pl.ds(0, RANGE * D)],
                        out_hbm.at[pl.ds(r0 * D, RANGE * D)])

    for k in range(4):
        r = wid + 32 * k

        @pl.when(r < NRANGES)
        def _():
            do_range(r)


def _edge_phase(xl, xr, c_pad, src_s, noff_pad, att):
    mesh = plsc.VectorSubcoreMesh(core_axis_name="c", subcore_axis_name="s")
    f = functools.partial(
        pl.kernel,
        mesh=mesh,
        out_type=jax.ShapeDtypeStruct((M * D,), jnp.float32),
        scratch_types=[
            pltpu.VMEM(((RANGE + 1) * D,), jnp.float32),
            pltpu.VMEM((CH, D), jnp.float32),
            pltpu.VMEM((CH,), jnp.int32),
            pltpu.VMEM((NSL,), jnp.int32),
            pltpu.VMEM((NSL,), jnp.float32),
            pltpu.VMEM((3 * D,), jnp.float32),
            pltpu.SemaphoreType.DMA,
        ],
    )(_edge_body)
    out = f(xl, xr.reshape(M * D), c_pad, src_s, noff_pad, att.reshape(3 * D))
    return out.reshape(M, D)


def _bb_body(e_ref, b5_ref, w1_ref, b1_ref, w2_ref, b2_ref, w3_ref, b3_ref,
             o_ref):
    e = e_ref[...] + b5_ref[...]
    h = jnp.maximum(e @ w1_ref[...] + b1_ref[...], 0.0)
    h = jnp.maximum(h @ w2_ref[...] + b2_ref[...], 0.0)
    o_ref[...] = h @ w3_ref[...] + b3_ref[...]


def _backbone(rows_pad, b5cat, bp):
    out = pl.pallas_call(
        _bb_body,
        out_shape=jax.ShapeDtypeStruct((8, 256), jnp.float32),
    )(rows_pad, b5cat, bp['W1'], bp['b1'].reshape(1, -1),
      bp['W2'], bp['b2'].reshape(1, -1), bp['W3'], bp['b3'].reshape(1, -1))
    return out[:1]


def kernel(x_lhs, edge_index_lhs, lhs_root, x_rhs, edge_index_rhs, rhs_root,
           x_sketch, edge_index_sketch, sketch_root, params):
    gp = [params['lhs'], params['rhs'], params['sketch']]

    loop = jnp.arange(N, dtype=jnp.int32)
    srcs = jnp.concatenate([
        edge_index_lhs[0].astype(jnp.int32),
        edge_index_rhs[0].astype(jnp.int32) + N,
        edge_index_sketch[0].astype(jnp.int32) + 2 * N,
        loop, loop + N, loop + 2 * N,
    ])
    dsts = jnp.concatenate([
        edge_index_lhs[1].astype(jnp.int32),
        edge_index_rhs[1].astype(jnp.int32) + N,
        edge_index_sketch[1].astype(jnp.int32) + 2 * N,
        loop, loop + N, loop + 2 * N,
    ])
    dst_s, src_s = lax.sort_key_val(dsts, srcs)
    noff = jnp.searchsorted(dst_s, jnp.arange(M + 1), side='left').astype(jnp.int32)
    src_pad = jnp.concatenate([src_s, jnp.zeros((EEP - EE,), jnp.int32)])
    noff_pad = jnp.concatenate(
        [noff, jnp.full((NOFF_PAD - (M + 1),), _SENTINEL, jnp.int32)])

    x0 = jnp.stack([x_lhs, x_rhs, x_sketch])

    h = x0
    b_prev = None
    for l in range(5):
        wl = jnp.stack([gp[g][l]['Wl'] for g in range(3)])
        wr = jnp.stack([gp[g][l]['Wr'] for g in range(3)])
        att = jnp.stack([gp[g][l]['att'] for g in range(3)])
        xl, xr, c = _layer_mm(h, b_prev, wl, wr, att)
        c_pad = jnp.concatenate([c, jnp.zeros((NOFF_PAD - M,), jnp.float32)])
        out = _edge_phase(xl, xr, c_pad, src_pad, noff_pad, att)
        h = out.reshape(3, N, D)
        b_prev = jnp.stack([gp[g][l]['b'] for g in range(3)])

    roots = jnp.stack([
        sketch_root[0].astype(jnp.int32) + 2 * N,
        lhs_root[0].astype(jnp.int32),
        rhs_root[0].astype(jnp.int32) + N,
    ])
    rows = out[roots]
    rows_pad = jnp.pad(rows.reshape(1, 3 * D), ((0, 7), (0, 0)))
    b5cat = jnp.concatenate(
        [gp[2][4]['b'], gp[0][4]['b'], gp[1][4]['b']]).reshape(1, 3 * D)
    return _backbone(rows_pad, b5cat, params['backbone'])


# speedup vs baseline: 5.6498x; 1.2054x over previous
import functools

import jax
import jax.numpy as jnp
from jax import lax
from jax.experimental import pallas as pl
from jax.experimental.pallas import tpu as pltpu
from jax.experimental.pallas import tpu_sc as plsc

N = 10000
M = 30000
D = 256
NJ = D // 16
RANGE = 240
NRANGES = M // RANGE
CH = 64
EE = 3 * (320000 + N)
EEP = 990080
NSL = 336
NOFF_PAD = M + NSL + 16

_SENTINEL = 1 << 30


def _mm_body_nobias(h_ref, wl_ref, wr_ref, att_ref, xl_ref, xr_ref, c_ref):
    x = h_ref[0]
    xl = jnp.dot(x, wl_ref[0], preferred_element_type=jnp.float32)
    xr = jnp.dot(x, wr_ref[0], preferred_element_type=jnp.float32)
    xl_ref[0] = xl
    xr_ref[0] = xr
    t = xl + xr
    e = jnp.maximum(t, 0.2 * t)
    c_ref[0, 0] = jnp.sum(e * att_ref[0], axis=1)


def _mm_body_bias(h_ref, b_ref, wl_ref, wr_ref, att_ref, xl_ref, xr_ref, c_ref):
    x = jnp.maximum(h_ref[0] + b_ref[0], 0.0)
    xl = jnp.dot(x, wl_ref[0], preferred_element_type=jnp.float32)
    xr = jnp.dot(x, wr_ref[0], preferred_element_type=jnp.float32)
    xl_ref[0] = xl
    xr_ref[0] = xr
    t = xl + xr
    e = jnp.maximum(t, 0.2 * t)
    c_ref[0, 0] = jnp.sum(e * att_ref[0], axis=1)


def _layer_mm(h, b_prev, wl, wr, att):
    din = h.shape[-1]
    nb = N // 400
    grid = (3, nb)
    h_spec = pl.BlockSpec((1, 400, din), lambda g, i: (g, i, 0))
    w_spec = pl.BlockSpec((1, din, D), lambda g, i: (g, 0, 0))
    att_spec = pl.BlockSpec((1, 1, D), lambda g, i: (g, 0, 0))
    b_spec = pl.BlockSpec((1, 1, D), lambda g, i: (g, 0, 0))
    out_specs = [
        pl.BlockSpec((1, 400, D), lambda g, i: (g, i, 0)),
        pl.BlockSpec((1, 400, D), lambda g, i: (g, i, 0)),
        pl.BlockSpec((1, 1, 400), lambda g, i: (g * nb + i, 0, 0)),
    ]
    out_shape = [
        jax.ShapeDtypeStruct((3, N, D), jnp.float32),
        jax.ShapeDtypeStruct((3, N, D), jnp.float32),
        jax.ShapeDtypeStruct((3 * nb, 1, 400), jnp.float32),
    ]
    att3 = att.reshape(3, 1, D)
    if b_prev is None:
        xl, xr, c = pl.pallas_call(
            _mm_body_nobias,
            grid=grid,
            in_specs=[h_spec, w_spec, w_spec, att_spec],
            out_specs=out_specs,
            out_shape=out_shape,
        )(h, wl, wr, att3)
    else:
        xl, xr, c = pl.pallas_call(
            _mm_body_bias,
            grid=grid,
            in_specs=[h_spec, b_spec, w_spec, w_spec, att_spec],
            out_specs=out_specs,
            out_shape=out_shape,
        )(h, b_prev.reshape(3, 1, D), wl, wr, att3)
    return xl.reshape(M, D), xr.reshape(M, D), c.reshape(M)


def _edge_body(xl_hbm, xr_hbm, c_hbm, src_hbm, noff_hbm, att_hbm, out_hbm,
               xr_v, xl_a, xl_b, src_a, src_b, noff_v, c_v, att_v,
               sem_a, sem_b):
    cid = lax.axis_index("c")
    sid = lax.axis_index("s")
    wid = sid * 2 + cid
    lanes = lax.iota(jnp.int32, 16)

    pltpu.sync_copy(att_hbm, att_v)

    def do_range(r):
        r0 = r * RANGE
        pltpu.sync_copy(xr_hbm.at[pl.ds(r0 * D, RANGE * D)],
                        xr_v.at[pl.ds(0, RANGE * D)])
        pltpu.sync_copy(noff_hbm.at[pl.ds(r0, NSL)], noff_v)
        pltpu.sync_copy(c_hbm.at[pl.ds(r0, NSL)], c_v)

        head = noff_v[pl.ds(0, 16)]
        estart = head[0]
        eend = noff_v[pl.ds(RANGE, 16)][0]
        c0 = lax.shift_right_logical(estart, 6)
        c1 = lax.shift_right_logical(eend, 6) + 1

        g0 = (r0 >= N).astype(jnp.int32) + (r0 >= 2 * N).astype(jnp.int32)
        zero16 = jnp.zeros((16,), jnp.float32)
        init = (r0, g0, head[1], c_v[pl.ds(0, 16)][0], zero16) + tuple(
            zero16 for _ in range(NJ))

        def make_edge_body(ebase, xl_v):
            def edge_body(e, ec):
                v, g, enext, cv, den = ec[0], ec[1], ec[2], ec[3], ec[4]
                accs = ec[5:]
                eg = ebase + e

                def fin(op):
                    v_, g_, enext_, cv_, den_ = op[0], op[1], op[2], op[3], op[4]
                    accs_ = op[5:]
                    invd = 1.0 / (den_ + 1e-16)
                    sidx = jnp.minimum(v_ - r0, RANGE)
                    sbase = sidx * D
                    for j in range(NJ):
                        xr_v[pl.ds(sbase + 16 * j, 16)] = accs_[j] * invd
                    v2 = v_ + 1
                    g2 = ((v2 >= N).astype(jnp.int32)
                          + (v2 >= 2 * N).astype(jnp.int32))
                    enext2 = noff_v[pl.ds(v2 - r0 + 1, 16)][0]
                    cv2 = c_v[pl.ds(v2 - r0, 16)][0]
                    return (v2, g2, enext2, cv2, zero16) + tuple(
                        zero16 for _ in range(NJ))

                ec = lax.cond(eg == enext, fin, lambda op: op, ec)
                v, g, enext, cv, den = ec[0], ec[1], ec[2], ec[3], ec[4]
                accs = ec[5:]

                xbase = jnp.minimum(v - r0, RANGE - 1) * D
                abase = g * D
                p = zero16
                xls = []
                for j in range(NJ):
                    xlj = xl_v[e, pl.ds(16 * j, 16)]
                    xls.append(xlj)
                    xrj = xr_v[pl.ds(xbase + 16 * j, 16)]
                    t = xlj + xrj
                    lr = jnp.maximum(t, 0.2 * t)
                    p = p + att_v[pl.ds(abase + 16 * j, 16)] * lr
                for k in (8, 4, 2, 1):
                    p = p + p.at[lanes ^ k].get(mode='promise_in_bounds')
                inr = jnp.logical_and(eg >= estart, eg < eend)
                wv = jnp.exp(p - cv)
                wv = jnp.where(inr, wv, 0.0)
                den = den + wv
                accs = tuple(accs[j] + wv * xls[j] for j in range(NJ))
                return (v, g, enext, cv, den) + accs

            return edge_body

        def gather(ci, src_ref, xl_ref, sem):
            pltpu.sync_copy(src_hbm.at[pl.ds(ci * CH, CH)], src_ref)
            pltpu.async_copy(xl_hbm.at[src_ref], xl_ref, sem)

        def wait_gather(src_ref, xl_ref, sem):
            pltpu.make_async_copy(xl_hbm.at[src_ref], xl_ref, sem).wait()

        def run_chunk(ci, xl_ref, carry):
            return lax.fori_loop(0, CH, make_edge_body(ci * CH, xl_ref),
                                 carry, unroll=4)

        nch = c1 - c0
        npf = lax.shift_right_logical(nch, 1)
        gather(c0, src_a, xl_a, sem_a)

        def pair_body(i, carry):
            ca = c0 + 2 * i
            gather(ca + 1, src_b, xl_b, sem_b)
            wait_gather(src_a, xl_a, sem_a)
            carry = run_chunk(ca, xl_a, carry)

            @pl.when(ca + 2 < c1)
            def _():
                gather(ca + 2, src_a, xl_a, sem_a)

            wait_gather(src_b, xl_b, sem_b)
            return run_chunk(ca + 1, xl_b, carry)

        carry = lax.fori_loop(0, npf, pair_body, init)

        @pl.when((nch & 1) == 1)
        def _():
            wait_gather(src_a, xl_a, sem_a)
            run_chunk(c1 - 1, xl_a, carry)

        pltpu.sync_copy(xr_v.at[pl.ds(0, RANGE * D)],
                        out_hbm.at[pl.ds(r0 * D, RANGE * D)])

    for k in range(4):
        r = wid + 32 * k

        @pl.when(r < NRANGES)
        def _():
            do_range(r)


def _edge_phase(xl, xr, c_pad, src_s, noff_pad, att):
    mesh = plsc.VectorSubcoreMesh(core_axis_name="c", subcore_axis_name="s")
    f = functools.partial(
        pl.kernel,
        mesh=mesh,
        out_type=jax.ShapeDtypeStruct((M * D,), jnp.float32),
        scratch_types=[
            pltpu.VMEM(((RANGE + 1) * D,), jnp.float32),
            pltpu.VMEM((CH, D), jnp.float32),
            pltpu.VMEM((CH, D), jnp.float32),
            pltpu.VMEM((CH,), jnp.int32),
            pltpu.VMEM((CH,), jnp.int32),
            pltpu.VMEM((NSL,), jnp.int32),
            pltpu.VMEM((NSL,), jnp.float32),
            pltpu.VMEM((3 * D,), jnp.float32),
            pltpu.SemaphoreType.DMA,
            pltpu.SemaphoreType.DMA,
        ],
    )(_edge_body)
    out = f(xl, xr.reshape(M * D), c_pad, src_s, noff_pad, att.reshape(3 * D))
    return out.reshape(M, D)


def _bb_body(e_ref, b5_ref, w1_ref, b1_ref, w2_ref, b2_ref, w3_ref, b3_ref,
             o_ref):
    e = e_ref[...] + b5_ref[...]
    h = jnp.maximum(e @ w1_ref[...] + b1_ref[...], 0.0)
    h = jnp.maximum(h @ w2_ref[...] + b2_ref[...], 0.0)
    o_ref[...] = h @ w3_ref[...] + b3_ref[...]


def _backbone(rows_pad, b5cat, bp):
    out = pl.pallas_call(
        _bb_body,
        out_shape=jax.ShapeDtypeStruct((8, 256), jnp.float32),
    )(rows_pad, b5cat, bp['W1'], bp['b1'].reshape(1, -1),
      bp['W2'], bp['b2'].reshape(1, -1), bp['W3'], bp['b3'].reshape(1, -1))
    return out[:1]


def kernel(x_lhs, edge_index_lhs, lhs_root, x_rhs, edge_index_rhs, rhs_root,
           x_sketch, edge_index_sketch, sketch_root, params):
    gp = [params['lhs'], params['rhs'], params['sketch']]

    loop = jnp.arange(N, dtype=jnp.int32)
    srcs = jnp.concatenate([
        edge_index_lhs[0].astype(jnp.int32),
        edge_index_rhs[0].astype(jnp.int32) + N,
        edge_index_sketch[0].astype(jnp.int32) + 2 * N,
        loop, loop + N, loop + 2 * N,
    ])
    dsts = jnp.concatenate([
        edge_index_lhs[1].astype(jnp.int32),
        edge_index_rhs[1].astype(jnp.int32) + N,
        edge_index_sketch[1].astype(jnp.int32) + 2 * N,
        loop, loop + N, loop + 2 * N,
    ])
    dst_s, src_s = lax.sort_key_val(dsts, srcs)
    noff = jnp.searchsorted(dst_s, jnp.arange(M + 1), side='left').astype(jnp.int32)
    src_pad = jnp.concatenate([src_s, jnp.zeros((EEP - EE,), jnp.int32)])
    noff_pad = jnp.concatenate(
        [noff, jnp.full((NOFF_PAD - (M + 1),), _SENTINEL, jnp.int32)])

    x0 = jnp.stack([x_lhs, x_rhs, x_sketch])

    h = x0
    b_prev = None
    for l in range(5):
        wl = jnp.stack([gp[g][l]['Wl'] for g in range(3)])
        wr = jnp.stack([gp[g][l]['Wr'] for g in range(3)])
        att = jnp.stack([gp[g][l]['att'] for g in range(3)])
        xl, xr, c = _layer_mm(h, b_prev, wl, wr, att)
        c_pad = jnp.concatenate([c, jnp.zeros((NOFF_PAD - M,), jnp.float32)])
        out = _edge_phase(xl, xr, c_pad, src_pad, noff_pad, att)
        h = out.reshape(3, N, D)
        b_prev = jnp.stack([gp[g][l]['b'] for g in range(3)])

    roots = jnp.stack([
        sketch_root[0].astype(jnp.int32) + 2 * N,
        lhs_root[0].astype(jnp.int32),
        rhs_root[0].astype(jnp.int32) + N,
    ])
    rows = out[roots]
    rows_pad = jnp.pad(rows.reshape(1, 3 * D), ((0, 7), (0, 0)))
    b5cat = jnp.concatenate(
        [gp[2][4]['b'], gp[0][4]['b'], gp[1][4]['b']]).reshape(1, 3 * D)
    return _backbone(rows_pad, b5cat, params['backbone'])
